# baseline (speedup 1.0000x reference)
import functools

import jax
import jax.numpy as jnp
from jax import lax
from jax.experimental import pallas as pl
from jax.experimental.pallas import tpu as pltpu
from jax.experimental.pallas import tpu_sc as plsc

_N = 10000
_E = 320000
_D = 128
_H = 64

_NSUB = 16
_EPT = _E // _NSUB
_C = 80
_NCH = _EPT // _C
_RN = 3456
_NRANGE = 3
_AROWS = _RN + 8
_JUNK = _RN
_FPT = _RN // _NSUB

_NBLK = 1000
_EBLK = 2000


def _node_prep_body(x_ref, wsg, wdg, wdu, wsu, bsg, bdg, bdu, bsu,
                    s_ref, du_ref, xu_ref):
    x = x_ref[...]
    dims = (((1,), (1,)), ((), ()))
    sg = lax.dot_general(x, wsg[...], dims, preferred_element_type=jnp.float32) + bsg[...]
    dg = lax.dot_general(x, wdg[...], dims, preferred_element_type=jnp.float32) + bdg[...]
    du = lax.dot_general(x, wdu[...], dims, preferred_element_type=jnp.float32) + bdu[...]
    su = lax.dot_general(x, wsu[...], dims, preferred_element_type=jnp.float32) + bsu[...]
    s_ref[...] = sg
    du_ref[0] = jnp.concatenate([dg[:, :_H], du[:, :_H]], axis=1)
    du_ref[1] = jnp.concatenate([dg[:, _H:], du[:, _H:]], axis=1)
    xu_ref[...] = su


def _edge_prep_body(ef_ref, weg, beg, eg_ref):
    dims = (((1,), (1,)), ((), ()))
    eg = lax.dot_general(ef_ref[...], weg[...], dims,
                         preferred_element_type=jnp.float32) + beg[...]
    eg_ref[0] = eg[:, :_H]
    eg_ref[1] = eg[:, _H:]


def _sc_edge_body(scat, ducat, eeg, ii_hbm, jj_hbm,
                  y_out, msig_out, ab_out, stats_out,
                  ii_raw, jj_adj, ii_s, sg_b, dd_b, ee_b, y_b, msf_b,
                  stat_b, zbuf, ab_sh, sem):
    c = lax.axis_index("c")
    s = lax.axis_index("s")
    cn = c * _N
    ch = c * _H

    zero16 = jnp.zeros((16,), jnp.float32)

    def zrow(r, carry):
        for q in range(_D // 16):
            zbuf[r, pl.ds(q * 16, 16)] = zero16
        return carry

    lax.fori_loop(0, _FPT, zrow, 0)
    for rr in range(8):
        for q in range(_H // 16):
            stat_b[rr, pl.ds(q * 16, 16)] = zero16

    pltpu.sync_copy(zbuf, ab_sh.at[pl.ds(s * _FPT, _FPT)])
    plsc.subcore_barrier()

    base0 = s * _EPT

    def flush_and_rezero(rng, rezero):
        plsc.subcore_barrier()
        pltpu.sync_copy(
            ab_sh.at[pl.ds(s * _FPT, _FPT)],
            ab_out.at[pl.ds((c * _NRANGE + rng) * _RN + s * _FPT, _FPT)])
        if rezero:
            plsc.subcore_barrier()
            pltpu.sync_copy(zbuf, ab_sh.at[pl.ds(s * _FPT, _FPT)])
            plsc.subcore_barrier()

    def chunk1(k, carry):
        base = base0 + k * _C
        pltpu.sync_copy(ii_hbm.at[pl.ds(base, _C)], ii_raw)
        pltpu.sync_copy(jj_hbm.at[pl.ds(base, _C)], jj_adj)
        for t in range(_C // 16):
            sl = pl.ds(t * 16, 16)
            v = ii_raw[sl]
            jj_adj[sl] = jj_adj[sl] + cn
            ii_s[sl] = jnp.where(v < _RN, v, _JUNK)
        pltpu.async_copy(scat.at[ii_raw], sg_b, sem).wait()
        pltpu.async_copy(ducat.at[jj_adj], dd_b, sem).wait()
        pltpu.sync_copy(eeg.at[pl.ds(c * _E + base, _C)], ee_b)

        def row(r, rcarry):
            for q in range(_H // 16):
                sl = pl.ds(q * 16, 16)
                yv = sg_b[r, pl.ds(ch + q * 16, 16)] + dd_b[r, sl] + ee_b[r, sl]
                y_b[r, sl] = yv
                sig = 1.0 / (1.0 + jnp.exp(-yv))
                msf_b[r, sl] = dd_b[r, pl.ds(_H + q * 16, 16)] * sig
                msf_b[r, pl.ds(_H + q * 16, 16)] = sig
                stat_b[0, sl] = stat_b[0, sl] + yv
                stat_b[1, sl] = stat_b[1, sl] + yv * yv
            return rcarry

        lax.fori_loop(0, _C, row, 0)
        pltpu.sync_copy(y_b, y_out.at[pl.ds(c * _E + base, _C)])
        pltpu.sync_copy(msf_b, msig_out.at[pl.ds(c * _E + base, _C)])
        pltpu.sync_copy(msf_b, ab_sh.at[ii_s], add=True)
        return carry

    lax.fori_loop(0, _NCH, chunk1, 0)

    pltpu.sync_copy(stat_b, stats_out.at[pl.ds((c * _NSUB + s) * 8, 8)])
    flush_and_rezero(0, True)

    for rng in range(1, _NRANGE):
        lo = rng * _RN

        def chunk2(k, carry):
            base = base0 + k * _C
            pltpu.sync_copy(ii_hbm.at[pl.ds(base, _C)], ii_raw)
            for t in range(_C // 16):
                sl = pl.ds(t * 16, 16)
                v = ii_raw[sl] - lo
                inr = jnp.logical_and(v >= 0, v < _RN)
                ii_s[sl] = jnp.where(inr, v, _JUNK)
            pltpu.sync_copy(msig_out.at[pl.ds(c * _E + base, _C)], msf_b)
            pltpu.sync_copy(msf_b, ab_sh.at[ii_s], add=True)
            return carry

        lax.fori_loop(0, _NCH, chunk2, 0)
        flush_and_rezero(rng, rng < _NRANGE - 1)


def _node_fin1_body(a_ref, b_ref, xu_ref, xpre_ref, nstats_ref, acc):
    i = pl.program_id(0)

    @pl.when(i == 0)
    def _():
        acc[...] = jnp.zeros((2, _D), jnp.float32)

    xp = xu_ref[...] + a_ref[...] / (b_ref[...] + 1e-6)
    xpre_ref[...] = xp
    acc[0:1, :] = acc[0:1, :] + jnp.sum(xp, axis=0, keepdims=True)
    acc[1:2, :] = acc[1:2, :] + jnp.sum(xp * xp, axis=0, keepdims=True)
    nstats_ref[...] = acc[...]


def _node_fin2_body(xpre_ref, nf_ref, nstats_ref, gamma_ref, beta_ref, out_ref):
    mean = nstats_ref[0:1, :] / _N
    var = nstats_ref[1:2, :] / _N - mean * mean
    xp = xpre_ref[...]
    xn = (xp - mean) * lax.rsqrt(var + 1e-5) * gamma_ref[...] + beta_ref[...]
    out_ref[...] = nf_ref[...] + xn * jax.nn.sigmoid(xn)


def _edge_fin_body(y_ref, ef_ref, stats_ref, gamma_ref, beta_ref, out_ref):
    st = stats_ref[...].reshape(2, _NSUB, 8, _H)
    sums = jnp.sum(st, axis=1)
    halves = []
    for cc in range(2):
        mean = (sums[cc, 0, :] / _E).reshape(1, _H)
        var = (sums[cc, 1, :] / _E).reshape(1, _H) - mean * mean
        yv = y_ref[cc]
        g = gamma_ref[:, cc * _H:(cc + 1) * _H]
        bt = beta_ref[:, cc * _H:(cc + 1) * _H]
        yn = (yv - mean) * lax.rsqrt(var + 1e-5) * g + bt
        halves.append(yn * jax.nn.sigmoid(yn))
    out_ref[...] = ef_ref[...] + jnp.concatenate(halves, axis=1)


@jax.jit
def kernel(node_feats, edge_feats, edge_index, W_sg, b_sg, W_dg, b_dg,
           W_eg, b_eg, W_su, b_su, W_du, b_du, gamma_n, beta_n,
           gamma_e, beta_e):
    f32 = jnp.float32
    b_sg2 = b_sg.reshape(1, _D)
    b_dg2 = b_dg.reshape(1, _D)
    b_du2 = b_du.reshape(1, _D)
    b_su2 = b_su.reshape(1, _D)
    b_eg2 = b_eg.reshape(1, _D)

    n_grid = _N // _NBLK
    s_tab, du_tab, xu_tab = pl.pallas_call(
        _node_prep_body,
        grid=(n_grid,),
        in_specs=[
            pl.BlockSpec((_NBLK, _D), lambda i: (i, 0)),
            pl.BlockSpec((_D, _D), lambda i: (0, 0)),
            pl.BlockSpec((_D, _D), lambda i: (0, 0)),
            pl.BlockSpec((_D, _D), lambda i: (0, 0)),
            pl.BlockSpec((_D, _D), lambda i: (0, 0)),
            pl.BlockSpec((1, _D), lambda i: (0, 0)),
            pl.BlockSpec((1, _D), lambda i: (0, 0)),
            pl.BlockSpec((1, _D), lambda i: (0, 0)),
            pl.BlockSpec((1, _D), lambda i: (0, 0)),
        ],
        out_specs=[
            pl.BlockSpec((_NBLK, _D), lambda i: (i, 0)),
            pl.BlockSpec((2, _NBLK, _D), lambda i: (0, i, 0)),
            pl.BlockSpec((_NBLK, _D), lambda i: (i, 0)),
        ],
        out_shape=[
            jax.ShapeDtypeStruct((_N, _D), f32),
            jax.ShapeDtypeStruct((2, _N, _D), f32),
            jax.ShapeDtypeStruct((_N, _D), f32),
        ],
    )(node_feats, W_sg, W_dg, W_du, W_su, b_sg2, b_dg2, b_du2, b_su2)

    e_grid = _E // _EBLK
    eg_tab = pl.pallas_call(
        _edge_prep_body,
        grid=(e_grid,),
        in_specs=[
            pl.BlockSpec((_EBLK, _D), lambda i: (i, 0)),
            pl.BlockSpec((_D, _D), lambda i: (0, 0)),
            pl.BlockSpec((1, _D), lambda i: (0, 0)),
        ],
        out_specs=[pl.BlockSpec((2, _EBLK, _H), lambda i: (0, i, 0))],
        out_shape=[jax.ShapeDtypeStruct((2, _E, _H), f32)],
    )(edge_feats, W_eg, b_eg2)[0]

    scat = s_tab
    ducat = du_tab.reshape(2 * _N, _D)
    eeg = eg_tab.reshape(2 * _E, _H)
    idx_i = edge_index[0]
    idx_j = edge_index[1]

    mesh = plsc.VectorSubcoreMesh(core_axis_name="c", subcore_axis_name="s")
    sc_edge = functools.partial(
        pl.kernel,
        mesh=mesh,
        compiler_params=pltpu.CompilerParams(needs_layout_passes=False),
        out_type=[
            jax.ShapeDtypeStruct((2 * _E, _H), f32),
            jax.ShapeDtypeStruct((2 * _E, _D), f32),
            jax.ShapeDtypeStruct((2 * _NRANGE * _RN, _D), f32),
            jax.ShapeDtypeStruct((2 * _NSUB * 8, _H), f32),
        ],
        scratch_types=[
            pltpu.VMEM((_C,), jnp.int32),
            pltpu.VMEM((_C,), jnp.int32),
            pltpu.VMEM((_C,), jnp.int32),
            pltpu.VMEM((_C, _D), f32),
            pltpu.VMEM((_C, _D), f32),
            pltpu.VMEM((_C, _H), f32),
            pltpu.VMEM((_C, _H), f32),
            pltpu.VMEM((_C, _D), f32),
            pltpu.VMEM((8, _H), f32),
            pltpu.VMEM((_FPT, _D), f32),
            pltpu.VMEM_SHARED((_AROWS, _D), f32),
            pltpu.SemaphoreType.DMA,
        ],
    )(_sc_edge_body)
    y_flat, _msig, ab_flat, sc_stats = sc_edge(scat, ducat, eeg, idx_i, idx_j)

    ab = ab_flat.reshape(2, _NRANGE * _RN, _D)[:, :_N, :]
    a_tab = jnp.concatenate([ab[0, :, :_H], ab[1, :, :_H]], axis=1)
    b_tab = jnp.concatenate([ab[0, :, _H:], ab[1, :, _H:]], axis=1)
    y_tab = y_flat.reshape(2, _E, _H)

    xpre, nstats = pl.pallas_call(
        _node_fin1_body,
        grid=(n_grid,),
        in_specs=[
            pl.BlockSpec((_NBLK, _D), lambda i: (i, 0)),
            pl.BlockSpec((_NBLK, _D), lambda i: (i, 0)),
            pl.BlockSpec((_NBLK, _D), lambda i: (i, 0)),
        ],
        out_specs=[
            pl.BlockSpec((_NBLK, _D), lambda i: (i, 0)),
            pl.BlockSpec((2, _D), lambda i: (0, 0)),
        ],
        out_shape=[
            jax.ShapeDtypeStruct((_N, _D), f32),
            jax.ShapeDtypeStruct((2, _D), f32),
        ],
        scratch_shapes=[pltpu.VMEM((2, _D), f32)],
    )(a_tab, b_tab, xu_tab)

    x_out = pl.pallas_call(
        _node_fin2_body,
        grid=(n_grid,),
        in_specs=[
            pl.BlockSpec((_NBLK, _D), lambda i: (i, 0)),
            pl.BlockSpec((_NBLK, _D), lambda i: (i, 0)),
            pl.BlockSpec((2, _D), lambda i: (0, 0)),
            pl.BlockSpec((1, _D), lambda i: (0, 0)),
            pl.BlockSpec((1, _D), lambda i: (0, 0)),
        ],
        out_specs=pl.BlockSpec((_NBLK, _D), lambda i: (i, 0)),
        out_shape=jax.ShapeDtypeStruct((_N, _D), f32),
    )(xpre, node_feats, nstats, gamma_n.reshape(1, _D), beta_n.reshape(1, _D))

    y_out = pl.pallas_call(
        _edge_fin_body,
        grid=(e_grid,),
        in_specs=[
            pl.BlockSpec((2, _EBLK, _H), lambda i: (0, i, 0)),
            pl.BlockSpec((_EBLK, _D), lambda i: (i, 0)),
            pl.BlockSpec((2 * _NSUB * 8, _H), lambda i: (0, 0)),
            pl.BlockSpec((1, _D), lambda i: (0, 0)),
            pl.BlockSpec((1, _D), lambda i: (0, 0)),
        ],
        out_specs=pl.BlockSpec((_EBLK, _D), lambda i: (i, 0)),
        out_shape=jax.ShapeDtypeStruct((_E, _D), f32),
    )(y_tab, edge_feats, sc_stats, gamma_e.reshape(1, _D), beta_e.reshape(1, _D))

    return (x_out, y_out)


# speedup vs baseline: 1.3602x; 1.3602x over previous
import functools

import jax
import jax.numpy as jnp
from jax import lax
from jax.experimental import pallas as pl
from jax.experimental.pallas import tpu as pltpu
from jax.experimental.pallas import tpu_sc as plsc

_N = 10000
_E = 320000
_D = 128
_H = 64

_NSUB = 16
_EPT = _E // _NSUB
_C = 80
_NCH = _EPT // _C
_RN = 3344
_NRANGE = 3
_AROWS = _RN + 8
_JUNK = _RN
_ZPT = 208

_NBLK = 1000
_EBLK = 2000


def _node_prep_body(x_ref, wsg, wdg, wdu, wsu, bsg, bdg, bdu, bsu,
                    s_ref, du_ref, xu_ref):
    x = x_ref[...]
    dims = (((1,), (1,)), ((), ()))
    sg = lax.dot_general(x, wsg[...], dims, preferred_element_type=jnp.float32) + bsg[...]
    dg = lax.dot_general(x, wdg[...], dims, preferred_element_type=jnp.float32) + bdg[...]
    du = lax.dot_general(x, wdu[...], dims, preferred_element_type=jnp.float32) + bdu[...]
    su = lax.dot_general(x, wsu[...], dims, preferred_element_type=jnp.float32) + bsu[...]
    s_ref[...] = sg
    du_ref[0] = jnp.concatenate([dg[:, :_H], du[:, :_H]], axis=1)
    du_ref[1] = jnp.concatenate([dg[:, _H:], du[:, _H:]], axis=1)
    xu_ref[...] = su


def _edge_prep_body(ef_ref, weg, beg, eg_ref):
    dims = (((1,), (1,)), ((), ()))
    eg = lax.dot_general(ef_ref[...], weg[...], dims,
                         preferred_element_type=jnp.float32) + beg[...]
    eg_ref[0] = eg[:, :_H]
    eg_ref[1] = eg[:, _H:]


def _sc_edge_body(scat, ducat, eeg, ii_hbm, jjc_hbm, iic_hbm,
                  y_out, msig_out, ab_out, stats_out,
                  ii0, ii1, jj0, jj1, is0, is1, sg0, sg1, dd0, dd1,
                  ee0, ee1, yy0, yy1, mf0, mf1, stat_b, ab_sh,
                  isem0, isem1, gsem0, gsem1, osem0, osem1):
    c = lax.axis_index("c")
    s = lax.axis_index("s")
    ch = c * _H
    II = (ii0, ii1)
    JJ = (jj0, jj1)
    IS = (is0, is1)
    SG = (sg0, sg1)
    DD = (dd0, dd1)
    EE = (ee0, ee1)
    YY = (yy0, yy1)
    MF = (mf0, mf1)
    ISEM = (isem0, isem1)
    GSEM = (gsem0, gsem1)
    OSEM = (osem0, osem1)

    zero16 = jnp.zeros((16,), jnp.float32)
    base0 = s * _EPT

    def zero_mf0():
        def zr(r, carry):
            for q in range(_D // 16):
                mf0[r, pl.ds(q * 16, 16)] = zero16
            return carry
        lax.fori_loop(0, _C, zr, 0)

    def zero_acc():
        off = s * _ZPT
        pltpu.sync_copy(mf0, ab_sh.at[pl.ds(off, _C)])
        pltpu.sync_copy(mf0, ab_sh.at[pl.ds(off + _C, _C)])
        pltpu.sync_copy(mf0.at[pl.ds(0, _ZPT - 2 * _C)],
                        ab_sh.at[pl.ds(off + 2 * _C, _ZPT - 2 * _C)])

        @pl.when(s == _NSUB - 1)
        def _():
            pltpu.sync_copy(mf0.at[pl.ds(0, _AROWS - _NSUB * _ZPT)],
                            ab_sh.at[pl.ds(_NSUB * _ZPT,
                                           _AROWS - _NSUB * _ZPT)])

    def flush(rng):
        obase = (c * _NRANGE + rng) * _RN
        pltpu.sync_copy(ab_sh.at[pl.ds(s * _ZPT, _ZPT)],
                        ab_out.at[pl.ds(obase + s * _ZPT, _ZPT)])

        @pl.when(s == _NSUB - 1)
        def _():
            pltpu.sync_copy(
                ab_sh.at[pl.ds(_NSUB * _ZPT, _RN - _NSUB * _ZPT)],
                ab_out.at[pl.ds(obase + _NSUB * _ZPT, _RN - _NSUB * _ZPT)])

    zero_mf0()
    for rr in range(8):
        for q in range(_H // 16):
            stat_b[rr, pl.ds(q * 16, 16)] = zero16
    zero_acc()
    plsc.subcore_barrier()

    def in_issue(k, b):
        base = base0 + k * _C
        pltpu.async_copy(ii_hbm.at[pl.ds(base, _C)], II[b], ISEM[b])
        pltpu.async_copy(jjc_hbm.at[pl.ds(c * _E + base, _C)], JJ[b], ISEM[b])
        pltpu.async_copy(eeg.at[pl.ds(c * _E + base, _C)], EE[b], ISEM[b])

    def in_wait(k, b):
        base = base0 + k * _C
        pltpu.make_async_copy(ii_hbm.at[pl.ds(base, _C)], II[b], ISEM[b]).wait()
        pltpu.make_async_copy(jjc_hbm.at[pl.ds(c * _E + base, _C)], JJ[b],
                              ISEM[b]).wait()
        pltpu.make_async_copy(eeg.at[pl.ds(c * _E + base, _C)], EE[b],
                              ISEM[b]).wait()

    def clamp(b):
        for t in range(_C // 16):
            sl = pl.ds(t * 16, 16)
            v = II[b][sl]
            IS[b][sl] = jnp.where(v < _RN, v, _JUNK)

    def g_issue(b):
        pltpu.async_copy(scat.at[II[b]], SG[b], GSEM[b])
        pltpu.async_copy(ducat.at[JJ[b]], DD[b], GSEM[b])

    def g_wait(b):
        pltpu.make_async_copy(scat.at[II[b]], SG[b], GSEM[b]).wait()
        pltpu.make_async_copy(ducat.at[JJ[b]], DD[b], GSEM[b]).wait()

    def out_issue(k, b):
        base = base0 + k * _C
        pltpu.async_copy(YY[b], y_out.at[pl.ds(c * _E + base, _C)], OSEM[b])
        pltpu.async_copy(MF[b], msig_out.at[pl.ds(c * _E + base, _C)], OSEM[b])

    def out_drain(k, b):
        base = base0 + k * _C
        pltpu.make_async_copy(YY[b], y_out.at[pl.ds(c * _E + base, _C)],
                              OSEM[b]).wait()
        pltpu.make_async_copy(MF[b], msig_out.at[pl.ds(c * _E + base, _C)],
                              OSEM[b]).wait()

    def compute(b):
        def row(r, rcarry):
            for q in range(_H // 16):
                sl = pl.ds(q * 16, 16)
                yv = (SG[b][r, pl.ds(ch + q * 16, 16)] + DD[b][r, sl]
                      + EE[b][r, sl])
                YY[b][r, sl] = yv
                sig = 1.0 / (1.0 + jnp.exp(-yv))
                MF[b][r, sl] = DD[b][r, pl.ds(_H + q * 16, 16)] * sig
                MF[b][r, pl.ds(_H + q * 16, 16)] = sig
                stat_b[0, sl] = stat_b[0, sl] + yv
                stat_b[1, sl] = stat_b[1, sl] + yv * yv
            return rcarry
        lax.fori_loop(0, _C, row, 0)

    in_issue(0, 0)
    in_wait(0, 0)
    clamp(0)
    g_issue(0)
    in_issue(1, 1)

    def body1(k2, carry):
        for b in (0, 1):
            k = 2 * k2 + b
            nb = 1 - b

            @pl.when(k >= 1)
            def _():
                out_drain(k - 1, nb)

            @pl.when(k + 1 < _NCH)
            def _():
                in_wait(k + 1, nb)
                clamp(nb)
                g_issue(nb)

            g_wait(b)
            compute(b)
            pltpu.sync_copy(MF[b], ab_sh.at[IS[b]], add=True)
            out_issue(k, b)

            @pl.when(k + 2 < _NCH)
            def _():
                in_issue(k + 2, b)
        return carry

    lax.fori_loop(0, _NCH // 2, body1, 0)
    out_drain(_NCH - 1, (_NCH - 1) % 2)

    pltpu.sync_copy(stat_b, stats_out.at[pl.ds((c * _NSUB + s) * 8, 8)])
    plsc.subcore_barrier()
    flush(0)
    plsc.subcore_barrier()
    zero_mf0()
    zero_acc()
    plsc.subcore_barrier()

    for rng in range(1, _NRANGE):
        ibase = (rng - 1) * _E

        def rin_issue(k, b):
            base = base0 + k * _C
            pltpu.async_copy(iic_hbm.at[pl.ds(ibase + base, _C)], IS[b],
                             ISEM[b])
            pltpu.async_copy(msig_out.at[pl.ds(c * _E + base, _C)], MF[b],
                             ISEM[b])

        def rin_wait(k, b):
            base = base0 + k * _C
            pltpu.make_async_copy(iic_hbm.at[pl.ds(ibase + base, _C)], IS[b],
                                  ISEM[b]).wait()
            pltpu.make_async_copy(msig_out.at[pl.ds(c * _E + base, _C)], MF[b],
                                  ISEM[b]).wait()

        rin_issue(0, 0)

        def body2(k2, carry):
            for b in (0, 1):
                k = 2 * k2 + b
                nb = 1 - b

                @pl.when(k + 1 < _NCH)
                def _():
                    rin_issue(k + 1, nb)

                rin_wait(k, b)
                pltpu.sync_copy(MF[b], ab_sh.at[IS[b]], add=True)
            return carry

        lax.fori_loop(0, _NCH // 2, body2, 0)
        plsc.subcore_barrier()
        flush(rng)
        if rng < _NRANGE - 1:
            plsc.subcore_barrier()
            zero_mf0()
            zero_acc()
            plsc.subcore_barrier()


def _node_fin1_body(a_ref, b_ref, xu_ref, xpre_ref, nstats_ref, acc):
    i = pl.program_id(0)

    @pl.when(i == 0)
    def _():
        acc[...] = jnp.zeros((2, _D), jnp.float32)

    xp = xu_ref[...] + a_ref[...] / (b_ref[...] + 1e-6)
    xpre_ref[...] = xp
    acc[0:1, :] = acc[0:1, :] + jnp.sum(xp, axis=0, keepdims=True)
    acc[1:2, :] = acc[1:2, :] + jnp.sum(xp * xp, axis=0, keepdims=True)
    nstats_ref[...] = acc[...]


def _node_fin2_body(xpre_ref, nf_ref, nstats_ref, gamma_ref, beta_ref, out_ref):
    mean = nstats_ref[0:1, :] / _N
    var = nstats_ref[1:2, :] / _N - mean * mean
    xp = xpre_ref[...]
    xn = (xp - mean) * lax.rsqrt(var + 1e-5) * gamma_ref[...] + beta_ref[...]
    out_ref[...] = nf_ref[...] + xn * jax.nn.sigmoid(xn)


def _edge_fin_body(y_ref, ef_ref, stats_ref, gamma_ref, beta_ref, out_ref):
    st = stats_ref[...].reshape(2, _NSUB, 8, _H)
    sums = jnp.sum(st, axis=1)
    halves = []
    for cc in range(2):
        mean = (sums[cc, 0, :] / _E).reshape(1, _H)
        var = (sums[cc, 1, :] / _E).reshape(1, _H) - mean * mean
        yv = y_ref[cc]
        g = gamma_ref[:, cc * _H:(cc + 1) * _H]
        bt = beta_ref[:, cc * _H:(cc + 1) * _H]
        yn = (yv - mean) * lax.rsqrt(var + 1e-5) * g + bt
        halves.append(yn * jax.nn.sigmoid(yn))
    out_ref[...] = ef_ref[...] + jnp.concatenate(halves, axis=1)


@jax.jit
def kernel(node_feats, edge_feats, edge_index, W_sg, b_sg, W_dg, b_dg,
           W_eg, b_eg, W_su, b_su, W_du, b_du, gamma_n, beta_n,
           gamma_e, beta_e):
    f32 = jnp.float32
    b_sg2 = b_sg.reshape(1, _D)
    b_dg2 = b_dg.reshape(1, _D)
    b_du2 = b_du.reshape(1, _D)
    b_su2 = b_su.reshape(1, _D)
    b_eg2 = b_eg.reshape(1, _D)

    n_grid = _N // _NBLK
    s_tab, du_tab, xu_tab = pl.pallas_call(
        _node_prep_body,
        grid=(n_grid,),
        in_specs=[
            pl.BlockSpec((_NBLK, _D), lambda i: (i, 0)),
            pl.BlockSpec((_D, _D), lambda i: (0, 0)),
            pl.BlockSpec((_D, _D), lambda i: (0, 0)),
            pl.BlockSpec((_D, _D), lambda i: (0, 0)),
            pl.BlockSpec((_D, _D), lambda i: (0, 0)),
            pl.BlockSpec((1, _D), lambda i: (0, 0)),
            pl.BlockSpec((1, _D), lambda i: (0, 0)),
            pl.BlockSpec((1, _D), lambda i: (0, 0)),
            pl.BlockSpec((1, _D), lambda i: (0, 0)),
        ],
        out_specs=[
            pl.BlockSpec((_NBLK, _D), lambda i: (i, 0)),
            pl.BlockSpec((2, _NBLK, _D), lambda i: (0, i, 0)),
            pl.BlockSpec((_NBLK, _D), lambda i: (i, 0)),
        ],
        out_shape=[
            jax.ShapeDtypeStruct((_N, _D), f32),
            jax.ShapeDtypeStruct((2, _N, _D), f32),
            jax.ShapeDtypeStruct((_N, _D), f32),
        ],
    )(node_feats, W_sg, W_dg, W_du, W_su, b_sg2, b_dg2, b_du2, b_su2)

    e_grid = _E // _EBLK
    eg_tab = pl.pallas_call(
        _edge_prep_body,
        grid=(e_grid,),
        in_specs=[
            pl.BlockSpec((_EBLK, _D), lambda i: (i, 0)),
            pl.BlockSpec((_D, _D), lambda i: (0, 0)),
            pl.BlockSpec((1, _D), lambda i: (0, 0)),
        ],
        out_specs=[pl.BlockSpec((2, _EBLK, _H), lambda i: (0, i, 0))],
        out_shape=[jax.ShapeDtypeStruct((2, _E, _H), f32)],
    )(edge_feats, W_eg, b_eg2)[0]

    scat = s_tab
    ducat = du_tab.reshape(2 * _N, _D)
    eeg = eg_tab.reshape(2 * _E, _H)
    idx_i = edge_index[0]
    idx_j = edge_index[1]
    jjcat = jnp.concatenate([idx_j, idx_j + _N])
    iic = []
    for rng in range(1, _NRANGE):
        lo = rng * _RN
        inr = jnp.logical_and(idx_i >= lo, idx_i < lo + _RN)
        iic.append(jnp.where(inr, idx_i - lo, _JUNK))
    iicat = jnp.concatenate(iic)

    mesh = plsc.VectorSubcoreMesh(core_axis_name="c", subcore_axis_name="s")
    sc_edge = functools.partial(
        pl.kernel,
        mesh=mesh,
        compiler_params=pltpu.CompilerParams(needs_layout_passes=False),
        out_type=[
            jax.ShapeDtypeStruct((2 * _E, _H), f32),
            jax.ShapeDtypeStruct((2 * _E, _D), f32),
            jax.ShapeDtypeStruct((2 * _NRANGE * _RN, _D), f32),
            jax.ShapeDtypeStruct((2 * _NSUB * 8, _H), f32),
        ],
        scratch_types=(
            [pltpu.VMEM((_C,), jnp.int32)] * 6
            + [pltpu.VMEM((_C, _D), f32)] * 4
            + [pltpu.VMEM((_C, _H), f32)] * 4
            + [pltpu.VMEM((_C, _D), f32)] * 2
            + [pltpu.VMEM((8, _H), f32)]
            + [pltpu.VMEM_SHARED((_AROWS, _D), f32)]
            + [pltpu.SemaphoreType.DMA] * 6
        ),
    )(_sc_edge_body)
    y_flat, _msig, ab_flat, sc_stats = sc_edge(
        scat, ducat, eeg, idx_i, jjcat, iicat)

    ab = ab_flat.reshape(2, _NRANGE * _RN, _D)[:, :_N, :]
    a_tab = jnp.concatenate([ab[0, :, :_H], ab[1, :, :_H]], axis=1)
    b_tab = jnp.concatenate([ab[0, :, _H:], ab[1, :, _H:]], axis=1)
    y_tab = y_flat.reshape(2, _E, _H)

    xpre, nstats = pl.pallas_call(
        _node_fin1_body,
        grid=(n_grid,),
        in_specs=[
            pl.BlockSpec((_NBLK, _D), lambda i: (i, 0)),
            pl.BlockSpec((_NBLK, _D), lambda i: (i, 0)),
            pl.BlockSpec((_NBLK, _D), lambda i: (i, 0)),
        ],
        out_specs=[
            pl.BlockSpec((_NBLK, _D), lambda i: (i, 0)),
            pl.BlockSpec((2, _D), lambda i: (0, 0)),
        ],
        out_shape=[
            jax.ShapeDtypeStruct((_N, _D), f32),
            jax.ShapeDtypeStruct((2, _D), f32),
        ],
        scratch_shapes=[pltpu.VMEM((2, _D), f32)],
    )(a_tab, b_tab, xu_tab)

    x_out = pl.pallas_call(
        _node_fin2_body,
        grid=(n_grid,),
        in_specs=[
            pl.BlockSpec((_NBLK, _D), lambda i: (i, 0)),
            pl.BlockSpec((_NBLK, _D), lambda i: (i, 0)),
            pl.BlockSpec((2, _D), lambda i: (0, 0)),
            pl.BlockSpec((1, _D), lambda i: (0, 0)),
            pl.BlockSpec((1, _D), lambda i: (0, 0)),
        ],
        out_specs=pl.BlockSpec((_NBLK, _D), lambda i: (i, 0)),
        out_shape=jax.ShapeDtypeStruct((_N, _D), f32),
    )(xpre, node_feats, nstats, gamma_n.reshape(1, _D), beta_n.reshape(1, _D))

    y_out = pl.pallas_call(
        _edge_fin_body,
        grid=(e_grid,),
        in_specs=[
            pl.BlockSpec((2, _EBLK, _H), lambda i: (0, i, 0)),
            pl.BlockSpec((_EBLK, _D), lambda i: (i, 0)),
            pl.BlockSpec((2 * _NSUB * 8, _H), lambda i: (0, 0)),
            pl.BlockSpec((1, _D), lambda i: (0, 0)),
            pl.BlockSpec((1, _D), lambda i: (0, 0)),
        ],
        out_specs=pl.BlockSpec((_EBLK, _D), lambda i: (i, 0)),
        out_shape=jax.ShapeDtypeStruct((_E, _D), f32),
    )(y_tab, edge_feats, sc_stats, gamma_e.reshape(1, _D), beta_e.reshape(1, _D))

    return (x_out, y_out)


# speedup vs baseline: 1.3844x; 1.0178x over previous
import functools

import jax
import jax.numpy as jnp
from jax import lax
from jax.experimental import pallas as pl
from jax.experimental.pallas import tpu as pltpu
from jax.experimental.pallas import tpu_sc as plsc

_N = 10000
_E = 320000
_D = 128
_H = 64

_NSUB = 16
_EPT = _E // _NSUB
_C = 80
_NCH = _EPT // _C
_RN = 3344
_NRANGE = 3
_AROWS = _RN + 8
_JUNK = _RN
_ZPT = 208

_NBLK = 1000
_EBLK = 2000


def _node_prep_body(x_ref, wsg, wdg, wdu, wsu, bsg, bdg, bdu, bsu,
                    s_ref, du_ref, xu_ref):
    x = x_ref[...]
    dims = (((1,), (1,)), ((), ()))
    sg = lax.dot_general(x, wsg[...], dims, preferred_element_type=jnp.float32) + bsg[...]
    dg = lax.dot_general(x, wdg[...], dims, preferred_element_type=jnp.float32) + bdg[...]
    du = lax.dot_general(x, wdu[...], dims, preferred_element_type=jnp.float32) + bdu[...]
    su = lax.dot_general(x, wsu[...], dims, preferred_element_type=jnp.float32) + bsu[...]
    s_ref[...] = sg
    du_ref[0] = jnp.concatenate([dg[:, :_H], du[:, :_H]], axis=1)
    du_ref[1] = jnp.concatenate([dg[:, _H:], du[:, _H:]], axis=1)
    xu_ref[...] = su


def _edge_prep_body(ef_ref, weg, beg, eg_ref):
    dims = (((1,), (1,)), ((), ()))
    eg = lax.dot_general(ef_ref[...], weg[...], dims,
                         preferred_element_type=jnp.float32) + beg[...]
    eg_ref[0] = eg[:, :_H]
    eg_ref[1] = eg[:, _H:]


def _sc_edge_body(scat, ducat, eeg, ii_hbm, jjc_hbm, iic_hbm,
                  y_out, msig_out, ab_out, stats_out,
                  ii0, ii1, jj0, jj1, is0, is1, sg0, sg1, dd0, dd1,
                  ee0, ee1, yy0, yy1, mf0, mf1, stat_b, ab_sh,
                  isem0, isem1, gsem0, gsem1, osem0, osem1):
    c = lax.axis_index("c")
    s = lax.axis_index("s")
    ch = c * _H
    II = (ii0, ii1)
    JJ = (jj0, jj1)
    IS = (is0, is1)
    SG = (sg0, sg1)
    DD = (dd0, dd1)
    EE = (ee0, ee1)
    YY = (yy0, yy1)
    MF = (mf0, mf1)
    ISEM = (isem0, isem1)
    GSEM = (gsem0, gsem1)
    OSEM = (osem0, osem1)

    zero16 = jnp.zeros((16,), jnp.float32)
    base0 = s * _EPT

    def zero_mf0():
        def zr(r, carry):
            for q in range(_D // 16):
                mf0[r, pl.ds(q * 16, 16)] = zero16
            return carry
        lax.fori_loop(0, _C, zr, 0)

    def zero_acc():
        off = s * _ZPT
        pltpu.sync_copy(mf0, ab_sh.at[pl.ds(off, _C)])
        pltpu.sync_copy(mf0, ab_sh.at[pl.ds(off + _C, _C)])
        pltpu.sync_copy(mf0.at[pl.ds(0, _ZPT - 2 * _C)],
                        ab_sh.at[pl.ds(off + 2 * _C, _ZPT - 2 * _C)])

        @pl.when(s == _NSUB - 1)
        def _():
            pltpu.sync_copy(mf0.at[pl.ds(0, _AROWS - _NSUB * _ZPT)],
                            ab_sh.at[pl.ds(_NSUB * _ZPT,
                                           _AROWS - _NSUB * _ZPT)])

    def flush(rng):
        obase = (c * _NRANGE + rng) * _RN
        pltpu.sync_copy(ab_sh.at[pl.ds(s * _ZPT, _ZPT)],
                        ab_out.at[pl.ds(obase + s * _ZPT, _ZPT)])

        @pl.when(s == _NSUB - 1)
        def _():
            pltpu.sync_copy(
                ab_sh.at[pl.ds(_NSUB * _ZPT, _RN - _NSUB * _ZPT)],
                ab_out.at[pl.ds(obase + _NSUB * _ZPT, _RN - _NSUB * _ZPT)])

    zero_mf0()
    for rr in range(8):
        for q in range(_H // 16):
            stat_b[rr, pl.ds(q * 16, 16)] = zero16
    zero_acc()
    plsc.subcore_barrier()

    def in_issue(k, b):
        base = base0 + k * _C
        pltpu.async_copy(ii_hbm.at[pl.ds(base, _C)], II[b], ISEM[b])
        pltpu.async_copy(jjc_hbm.at[pl.ds(c * _E + base, _C)], JJ[b], ISEM[b])
        pltpu.async_copy(eeg.at[pl.ds(c * _E + base, _C)], EE[b], ISEM[b])

    def in_wait(k, b):
        base = base0 + k * _C
        pltpu.make_async_copy(ii_hbm.at[pl.ds(base, _C)], II[b], ISEM[b]).wait()
        pltpu.make_async_copy(jjc_hbm.at[pl.ds(c * _E + base, _C)], JJ[b],
                              ISEM[b]).wait()
        pltpu.make_async_copy(eeg.at[pl.ds(c * _E + base, _C)], EE[b],
                              ISEM[b]).wait()

    def clamp(b):
        for t in range(_C // 16):
            sl = pl.ds(t * 16, 16)
            v = II[b][sl]
            IS[b][sl] = jnp.where(v < _RN, v, _JUNK)

    def g_issue(b):
        pltpu.async_copy(scat.at[II[b]], SG[b], GSEM[b])
        pltpu.async_copy(ducat.at[JJ[b]], DD[b], GSEM[b])

    def g_wait(b):
        pltpu.make_async_copy(scat.at[II[b]], SG[b], GSEM[b]).wait()
        pltpu.make_async_copy(ducat.at[JJ[b]], DD[b], GSEM[b]).wait()

    def out_issue(k, b):
        base = base0 + k * _C
        pltpu.async_copy(YY[b], y_out.at[pl.ds(c * _E + base, _C)], OSEM[b])
        pltpu.async_copy(MF[b], msig_out.at[pl.ds(c * _E + base, _C)], OSEM[b])

    def out_drain(k, b):
        base = base0 + k * _C
        pltpu.make_async_copy(YY[b], y_out.at[pl.ds(c * _E + base, _C)],
                              OSEM[b]).wait()
        pltpu.make_async_copy(MF[b], msig_out.at[pl.ds(c * _E + base, _C)],
                              OSEM[b]).wait()

    def compute(b):
        def row(r, rcarry):
            for q in range(_H // 16):
                sl = pl.ds(q * 16, 16)
                yv = (SG[b][r, pl.ds(ch + q * 16, 16)] + DD[b][r, sl]
                      + EE[b][r, sl])
                YY[b][r, sl] = yv
                sig = 1.0 / (1.0 + jnp.exp(-yv))
                MF[b][r, sl] = DD[b][r, pl.ds(_H + q * 16, 16)] * sig
                MF[b][r, pl.ds(_H + q * 16, 16)] = sig
                stat_b[0, sl] = stat_b[0, sl] + yv
                stat_b[1, sl] = stat_b[1, sl] + yv * yv
            return rcarry
        lax.fori_loop(0, _C, row, 0)

    in_issue(0, 0)
    in_wait(0, 0)
    clamp(0)
    g_issue(0)
    in_issue(1, 1)

    def body1(k2, carry):
        k0 = 2 * k2
        k1 = k0 + 1

        @pl.when(k0 >= 1)
        def _():
            out_drain(k0 - 1, 1)

        in_wait(k1, 1)
        clamp(1)
        g_issue(1)
        g_wait(0)
        compute(0)
        ds0 = pltpu.async_copy(MF[0], ab_sh.at[IS[0]], GSEM[0], add=True)
        out_issue(k0, 0)

        @pl.when(k0 + 2 < _NCH)
        def _():
            in_issue(k0 + 2, 0)

        out_drain(k0, 0)
        ds0.wait()

        @pl.when(k1 + 1 < _NCH)
        def _():
            in_wait(k1 + 1, 0)
            clamp(0)
            g_issue(0)

        g_wait(1)
        compute(1)
        ds1 = pltpu.async_copy(MF[1], ab_sh.at[IS[1]], GSEM[1], add=True)
        out_issue(k1, 1)

        @pl.when(k1 + 2 < _NCH)
        def _():
            in_issue(k1 + 2, 1)

        ds1.wait()
        return carry

    lax.fori_loop(0, _NCH // 2, body1, 0)
    out_drain(_NCH - 1, (_NCH - 1) % 2)

    pltpu.sync_copy(stat_b, stats_out.at[pl.ds((c * _NSUB + s) * 8, 8)])
    plsc.subcore_barrier()
    flush(0)
    plsc.subcore_barrier()
    zero_mf0()
    zero_acc()
    plsc.subcore_barrier()

    for rng in range(1, _NRANGE):
        ibase = (rng - 1) * _E

        def rin_issue(k, b):
            base = base0 + k * _C
            pltpu.async_copy(iic_hbm.at[pl.ds(ibase + base, _C)], IS[b],
                             ISEM[b])
            pltpu.async_copy(msig_out.at[pl.ds(c * _E + base, _C)], MF[b],
                             ISEM[b])

        def rin_wait(k, b):
            base = base0 + k * _C
            pltpu.make_async_copy(iic_hbm.at[pl.ds(ibase + base, _C)], IS[b],
                                  ISEM[b]).wait()
            pltpu.make_async_copy(msig_out.at[pl.ds(c * _E + base, _C)], MF[b],
                                  ISEM[b]).wait()

        rin_issue(0, 0)
        rin_issue(1, 1)

        def body2(k2, carry):
            k0 = 2 * k2
            k1 = k0 + 1
            rin_wait(k0, 0)
            d0 = pltpu.async_copy(MF[0], ab_sh.at[IS[0]], OSEM[0], add=True)
            rin_wait(k1, 1)
            d1 = pltpu.async_copy(MF[1], ab_sh.at[IS[1]], OSEM[1], add=True)
            d0.wait()

            @pl.when(k0 + 2 < _NCH)
            def _():
                rin_issue(k0 + 2, 0)

            d1.wait()

            @pl.when(k1 + 2 < _NCH)
            def _():
                rin_issue(k1 + 2, 1)
            return carry

        lax.fori_loop(0, _NCH // 2, body2, 0)
        plsc.subcore_barrier()
        flush(rng)
        if rng < _NRANGE - 1:
            plsc.subcore_barrier()
            zero_mf0()
            zero_acc()
            plsc.subcore_barrier()


def _node_fin1_body(a_ref, b_ref, xu_ref, xpre_ref, nstats_ref, acc):
    i = pl.program_id(0)

    @pl.when(i == 0)
    def _():
        acc[...] = jnp.zeros((2, _D), jnp.float32)

    xp = xu_ref[...] + a_ref[...] / (b_ref[...] + 1e-6)
    xpre_ref[...] = xp
    acc[0:1, :] = acc[0:1, :] + jnp.sum(xp, axis=0, keepdims=True)
    acc[1:2, :] = acc[1:2, :] + jnp.sum(xp * xp, axis=0, keepdims=True)
    nstats_ref[...] = acc[...]


def _node_fin2_body(xpre_ref, nf_ref, nstats_ref, gamma_ref, beta_ref, out_ref):
    mean = nstats_ref[0:1, :] / _N
    var = nstats_ref[1:2, :] / _N - mean * mean
    xp = xpre_ref[...]
    xn = (xp - mean) * lax.rsqrt(var + 1e-5) * gamma_ref[...] + beta_ref[...]
    out_ref[...] = nf_ref[...] + xn * jax.nn.sigmoid(xn)


def _edge_fin_body(y_ref, ef_ref, stats_ref, gamma_ref, beta_ref, out_ref):
    st = stats_ref[...].reshape(2, _NSUB, 8, _H)
    sums = jnp.sum(st, axis=1)
    halves = []
    for cc in range(2):
        mean = (sums[cc, 0, :] / _E).reshape(1, _H)
        var = (sums[cc, 1, :] / _E).reshape(1, _H) - mean * mean
        yv = y_ref[cc]
        g = gamma_ref[:, cc * _H:(cc + 1) * _H]
        bt = beta_ref[:, cc * _H:(cc + 1) * _H]
        yn = (yv - mean) * lax.rsqrt(var + 1e-5) * g + bt
        halves.append(yn * jax.nn.sigmoid(yn))
    out_ref[...] = ef_ref[...] + jnp.concatenate(halves, axis=1)


@jax.jit
def kernel(node_feats, edge_feats, edge_index, W_sg, b_sg, W_dg, b_dg,
           W_eg, b_eg, W_su, b_su, W_du, b_du, gamma_n, beta_n,
           gamma_e, beta_e):
    f32 = jnp.float32
    b_sg2 = b_sg.reshape(1, _D)
    b_dg2 = b_dg.reshape(1, _D)
    b_du2 = b_du.reshape(1, _D)
    b_su2 = b_su.reshape(1, _D)
    b_eg2 = b_eg.reshape(1, _D)

    n_grid = _N // _NBLK
    s_tab, du_tab, xu_tab = pl.pallas_call(
        _node_prep_body,
        grid=(n_grid,),
        in_specs=[
            pl.BlockSpec((_NBLK, _D), lambda i: (i, 0)),
            pl.BlockSpec((_D, _D), lambda i: (0, 0)),
            pl.BlockSpec((_D, _D), lambda i: (0, 0)),
            pl.BlockSpec((_D, _D), lambda i: (0, 0)),
            pl.BlockSpec((_D, _D), lambda i: (0, 0)),
            pl.BlockSpec((1, _D), lambda i: (0, 0)),
            pl.BlockSpec((1, _D), lambda i: (0, 0)),
            pl.BlockSpec((1, _D), lambda i: (0, 0)),
            pl.BlockSpec((1, _D), lambda i: (0, 0)),
        ],
        out_specs=[
            pl.BlockSpec((_NBLK, _D), lambda i: (i, 0)),
            pl.BlockSpec((2, _NBLK, _D), lambda i: (0, i, 0)),
            pl.BlockSpec((_NBLK, _D), lambda i: (i, 0)),
        ],
        out_shape=[
            jax.ShapeDtypeStruct((_N, _D), f32),
            jax.ShapeDtypeStruct((2, _N, _D), f32),
            jax.ShapeDtypeStruct((_N, _D), f32),
        ],
    )(node_feats, W_sg, W_dg, W_du, W_su, b_sg2, b_dg2, b_du2, b_su2)

    e_grid = _E // _EBLK
    eg_tab = pl.pallas_call(
        _edge_prep_body,
        grid=(e_grid,),
        in_specs=[
            pl.BlockSpec((_EBLK, _D), lambda i: (i, 0)),
            pl.BlockSpec((_D, _D), lambda i: (0, 0)),
            pl.BlockSpec((1, _D), lambda i: (0, 0)),
        ],
        out_specs=[pl.BlockSpec((2, _EBLK, _H), lambda i: (0, i, 0))],
        out_shape=[jax.ShapeDtypeStruct((2, _E, _H), f32)],
    )(edge_feats, W_eg, b_eg2)[0]

    scat = s_tab
    ducat = du_tab.reshape(2 * _N, _D)
    eeg = eg_tab.reshape(2 * _E, _H)
    idx_i = edge_index[0]
    idx_j = edge_index[1]
    jjcat = jnp.concatenate([idx_j, idx_j + _N])
    iic = []
    for rng in range(1, _NRANGE):
        lo = rng * _RN
        inr = jnp.logical_and(idx_i >= lo, idx_i < lo + _RN)
        iic.append(jnp.where(inr, idx_i - lo, _JUNK))
    iicat = jnp.concatenate(iic)

    mesh = plsc.VectorSubcoreMesh(core_axis_name="c", subcore_axis_name="s")
    sc_edge = functools.partial(
        pl.kernel,
        mesh=mesh,
        compiler_params=pltpu.CompilerParams(needs_layout_passes=False),
        out_type=[
            jax.ShapeDtypeStruct((2 * _E, _H), f32),
            jax.ShapeDtypeStruct((2 * _E, _D), f32),
            jax.ShapeDtypeStruct((2 * _NRANGE * _RN, _D), f32),
            jax.ShapeDtypeStruct((2 * _NSUB * 8, _H), f32),
        ],
        scratch_types=(
            [pltpu.VMEM((_C,), jnp.int32)] * 6
            + [pltpu.VMEM((_C, _D), f32)] * 4
            + [pltpu.VMEM((_C, _H), f32)] * 4
            + [pltpu.VMEM((_C, _D), f32)] * 2
            + [pltpu.VMEM((8, _H), f32)]
            + [pltpu.VMEM_SHARED((_AROWS, _D), f32)]
            + [pltpu.SemaphoreType.DMA] * 6
        ),
    )(_sc_edge_body)
    y_flat, _msig, ab_flat, sc_stats = sc_edge(
        scat, ducat, eeg, idx_i, jjcat, iicat)

    ab = ab_flat.reshape(2, _NRANGE * _RN, _D)[:, :_N, :]
    a_tab = jnp.concatenate([ab[0, :, :_H], ab[1, :, :_H]], axis=1)
    b_tab = jnp.concatenate([ab[0, :, _H:], ab[1, :, _H:]], axis=1)
    y_tab = y_flat.reshape(2, _E, _H)

    xpre, nstats = pl.pallas_call(
        _node_fin1_body,
        grid=(n_grid,),
        in_specs=[
            pl.BlockSpec((_NBLK, _D), lambda i: (i, 0)),
            pl.BlockSpec((_NBLK, _D), lambda i: (i, 0)),
            pl.BlockSpec((_NBLK, _D), lambda i: (i, 0)),
        ],
        out_specs=[
            pl.BlockSpec((_NBLK, _D), lambda i: (i, 0)),
            pl.BlockSpec((2, _D), lambda i: (0, 0)),
        ],
        out_shape=[
            jax.ShapeDtypeStruct((_N, _D), f32),
            jax.ShapeDtypeStruct((2, _D), f32),
        ],
        scratch_shapes=[pltpu.VMEM((2, _D), f32)],
    )(a_tab, b_tab, xu_tab)

    x_out = pl.pallas_call(
        _node_fin2_body,
        grid=(n_grid,),
        in_specs=[
            pl.BlockSpec((_NBLK, _D), lambda i: (i, 0)),
            pl.BlockSpec((_NBLK, _D), lambda i: (i, 0)),
            pl.BlockSpec((2, _D), lambda i: (0, 0)),
            pl.BlockSpec((1, _D), lambda i: (0, 0)),
            pl.BlockSpec((1, _D), lambda i: (0, 0)),
        ],
        out_specs=pl.BlockSpec((_NBLK, _D), lambda i: (i, 0)),
        out_shape=jax.ShapeDtypeStruct((_N, _D), f32),
    )(xpre, node_feats, nstats, gamma_n.reshape(1, _D), beta_n.reshape(1, _D))

    y_out = pl.pallas_call(
        _edge_fin_body,
        grid=(e_grid,),
        in_specs=[
            pl.BlockSpec((2, _EBLK, _H), lambda i: (0, i, 0)),
            pl.BlockSpec((_EBLK, _D), lambda i: (i, 0)),
            pl.BlockSpec((2 * _NSUB * 8, _H), lambda i: (0, 0)),
            pl.BlockSpec((1, _D), lambda i: (0, 0)),
            pl.BlockSpec((1, _D), lambda i: (0, 0)),
        ],
        out_specs=pl.BlockSpec((_EBLK, _D), lambda i: (i, 0)),
        out_shape=jax.ShapeDtypeStruct((_E, _D), f32),
    )(y_tab, edge_feats, sc_stats, gamma_e.reshape(1, _D), beta_e.reshape(1, _D))

    return (x_out, y_out)


# speedup vs baseline: 1.4748x; 1.0653x over previous
import functools

import jax
import jax.numpy as jnp
from jax import lax
from jax.experimental import pallas as pl
from jax.experimental.pallas import tpu as pltpu
from jax.experimental.pallas import tpu_sc as plsc

_N = 10000
_E = 320000
_D = 128
_H = 64

_NSUB = 16
_EPT = _E // _NSUB
_C = 80
_NCH = _EPT // _C
_RN = 3344
_NRANGE = 3
_AROWS = _RN + 8
_JUNK = _RN
_ZPT = 208

_NBLK = 1000
_EBLK = 2000


def _node_prep_body(x_ref, wsg, wdg, wdu, wsu, bsg, bdg, bdu, bsu,
                    s_ref, du_ref, xu_ref):
    x = x_ref[...]
    dims = (((1,), (1,)), ((), ()))
    sg = lax.dot_general(x, wsg[...], dims, preferred_element_type=jnp.float32) + bsg[...]
    dg = lax.dot_general(x, wdg[...], dims, preferred_element_type=jnp.float32) + bdg[...]
    du = lax.dot_general(x, wdu[...], dims, preferred_element_type=jnp.float32) + bdu[...]
    su = lax.dot_general(x, wsu[...], dims, preferred_element_type=jnp.float32) + bsu[...]
    s_ref[...] = sg
    du_ref[0] = jnp.concatenate([dg[:, :_H], du[:, :_H]], axis=1)
    du_ref[1] = jnp.concatenate([dg[:, _H:], du[:, _H:]], axis=1)
    xu_ref[...] = su


def _edge_prep_body(ef_ref, weg, beg, eg_ref):
    dims = (((1,), (1,)), ((), ()))
    eg = lax.dot_general(ef_ref[...], weg[...], dims,
                         preferred_element_type=jnp.float32) + beg[...]
    eg_ref[0] = eg[:, :_H]
    eg_ref[1] = eg[:, _H:]


def _sc_edge_body(scat, ducat, eeg, ii_hbm, jjc_hbm, iic_hbm,
                  y_out, msig_out, ab_out, stats_out,
                  ii0, ii1, jj0, jj1, is0, is1, sg0, sg1, dd0, dd1,
                  ee0, ee1, yy0, yy1, mf0, mf1, stat_b, ab_sh,
                  isem0, isem1, gsem0, gsem1, osem0, osem1):
    c = lax.axis_index("c")
    s = lax.axis_index("s")
    ch = c * _H
    II = (ii0, ii1)
    JJ = (jj0, jj1)
    IS = (is0, is1)
    SG = (sg0, sg1)
    DD = (dd0, dd1)
    EE = (ee0, ee1)
    YY = (yy0, yy1)
    MF = (mf0, mf1)
    ISEM = (isem0, isem1)
    GSEM = (gsem0, gsem1)
    OSEM = (osem0, osem1)

    zero16 = jnp.zeros((16,), jnp.float32)
    base0 = s * _EPT

    def zero_mf0():
        def zr(r, carry):
            for q in range(_D // 16):
                mf0[r, pl.ds(q * 16, 16)] = zero16
            return carry
        lax.fori_loop(0, _C, zr, 0)

    def zero_acc():
        off = s * _ZPT
        pltpu.sync_copy(mf0, ab_sh.at[pl.ds(off, _C)])
        pltpu.sync_copy(mf0, ab_sh.at[pl.ds(off + _C, _C)])
        pltpu.sync_copy(mf0.at[pl.ds(0, _ZPT - 2 * _C)],
                        ab_sh.at[pl.ds(off + 2 * _C, _ZPT - 2 * _C)])

        @pl.when(s == _NSUB - 1)
        def _():
            pltpu.sync_copy(mf0.at[pl.ds(0, _AROWS - _NSUB * _ZPT)],
                            ab_sh.at[pl.ds(_NSUB * _ZPT,
                                           _AROWS - _NSUB * _ZPT)])

    def flush(rng):
        obase = c * _RN
        pltpu.sync_copy(ab_sh.at[pl.ds(s * _ZPT, _ZPT)],
                        ab_out.at[pl.ds(obase + s * _ZPT, _ZPT)])

        @pl.when(s == _NSUB - 1)
        def _():
            pltpu.sync_copy(
                ab_sh.at[pl.ds(_NSUB * _ZPT, _RN - _NSUB * _ZPT)],
                ab_out.at[pl.ds(obase + _NSUB * _ZPT, _RN - _NSUB * _ZPT)])

    zero_mf0()
    for rr in range(8):
        for q in range(_H // 16):
            stat_b[rr, pl.ds(q * 16, 16)] = zero16
    zero_acc()
    plsc.subcore_barrier()

    def in_issue(k, b):
        base = base0 + k * _C
        pltpu.async_copy(ii_hbm.at[pl.ds(base, _C)], II[b], ISEM[b])
        pltpu.async_copy(jjc_hbm.at[pl.ds(c * _E + base, _C)], JJ[b], ISEM[b])
        pltpu.async_copy(eeg.at[pl.ds(c * _E + base, _C)], EE[b], ISEM[b])

    def in_wait(k, b):
        base = base0 + k * _C
        pltpu.make_async_copy(ii_hbm.at[pl.ds(base, _C)], II[b], ISEM[b]).wait()
        pltpu.make_async_copy(jjc_hbm.at[pl.ds(c * _E + base, _C)], JJ[b],
                              ISEM[b]).wait()
        pltpu.make_async_copy(eeg.at[pl.ds(c * _E + base, _C)], EE[b],
                              ISEM[b]).wait()

    def clamp(b):
        for t in range(_C // 16):
            sl = pl.ds(t * 16, 16)
            v = II[b][sl]
            IS[b][sl] = jnp.where(v < _RN, v, _JUNK)

    def g_issue(b):
        pltpu.async_copy(scat.at[II[b]], SG[b], GSEM[b])
        pltpu.async_copy(ducat.at[JJ[b]], DD[b], GSEM[b])

    def g_wait(b):
        pltpu.make_async_copy(scat.at[II[b]], SG[b], GSEM[b]).wait()
        pltpu.make_async_copy(ducat.at[JJ[b]], DD[b], GSEM[b]).wait()

    def out_issue(k, b):
        base = base0 + k * _C
        pltpu.async_copy(YY[b], y_out.at[pl.ds(c * _E + base, _C)], OSEM[b])
        pltpu.async_copy(MF[b], msig_out.at[pl.ds(c * _E + base, _C)], OSEM[b])

    def out_drain(k, b):
        base = base0 + k * _C
        pltpu.make_async_copy(YY[b], y_out.at[pl.ds(c * _E + base, _C)],
                              OSEM[b]).wait()
        pltpu.make_async_copy(MF[b], msig_out.at[pl.ds(c * _E + base, _C)],
                              OSEM[b]).wait()

    def compute(b):
        def row(r, rcarry):
            for q in range(_H // 16):
                sl = pl.ds(q * 16, 16)
                yv = (SG[b][r, pl.ds(ch + q * 16, 16)] + DD[b][r, sl]
                      + EE[b][r, sl])
                YY[b][r, sl] = yv
                sig = 1.0 / (1.0 + jnp.exp(-yv))
                MF[b][r, sl] = DD[b][r, pl.ds(_H + q * 16, 16)] * sig
                MF[b][r, pl.ds(_H + q * 16, 16)] = sig
                stat_b[0, sl] = stat_b[0, sl] + yv
                stat_b[1, sl] = stat_b[1, sl] + yv * yv
            return rcarry
        lax.fori_loop(0, _C, row, 0)

    in_issue(0, 0)
    in_wait(0, 0)
    clamp(0)
    g_issue(0)
    in_issue(1, 1)

    def body1(k2, carry):
        k0 = 2 * k2
        k1 = k0 + 1

        @pl.when(k0 >= 1)
        def _():
            out_drain(k0 - 1, 1)

        in_wait(k1, 1)
        clamp(1)
        g_issue(1)
        g_wait(0)
        compute(0)
        ds0 = pltpu.async_copy(MF[0], ab_sh.at[IS[0]], GSEM[0], add=True)
        out_issue(k0, 0)

        @pl.when(k0 + 2 < _NCH)
        def _():
            in_issue(k0 + 2, 0)

        out_drain(k0, 0)
        ds0.wait()

        @pl.when(k1 + 1 < _NCH)
        def _():
            in_wait(k1 + 1, 0)
            clamp(0)
            g_issue(0)

        g_wait(1)
        compute(1)
        ds1 = pltpu.async_copy(MF[1], ab_sh.at[IS[1]], GSEM[1], add=True)
        out_issue(k1, 1)

        @pl.when(k1 + 2 < _NCH)
        def _():
            in_issue(k1 + 2, 1)

        ds1.wait()
        return carry

    lax.fori_loop(0, _NCH // 2, body1, 0)
    out_drain(_NCH - 1, (_NCH - 1) % 2)

    pltpu.sync_copy(stat_b, stats_out.at[pl.ds((c * _NSUB + s) * 8, 8)])
    plsc.subcore_barrier()
    flush(0)


def _sc_relay_body(msig_hbm, iic_hbm,
                   ab_out, is0, is1, mf0, mf1, zb, ab_sh,
                   isem0, isem1, osem0, osem1):
    c = lax.axis_index("c")
    s = lax.axis_index("s")
    IS = (is0, is1)
    MF = (mf0, mf1)
    ISEM = (isem0, isem1)
    OSEM = (osem0, osem1)

    zero16 = jnp.zeros((16,), jnp.float32)
    base0 = s * _EPT

    def zr(r, carry):
        for q in range(_D // 16):
            zb[r, pl.ds(q * 16, 16)] = zero16
        return carry

    lax.fori_loop(0, _C, zr, 0)

    def zero_acc():
        off = s * _ZPT
        pltpu.sync_copy(zb, ab_sh.at[pl.ds(off, _C)])
        pltpu.sync_copy(zb, ab_sh.at[pl.ds(off + _C, _C)])
        pltpu.sync_copy(zb.at[pl.ds(0, _ZPT - 2 * _C)],
                        ab_sh.at[pl.ds(off + 2 * _C, _ZPT - 2 * _C)])

        @pl.when(s == _NSUB - 1)
        def _():
            pltpu.sync_copy(zb.at[pl.ds(0, _AROWS - _NSUB * _ZPT)],
                            ab_sh.at[pl.ds(_NSUB * _ZPT,
                                           _AROWS - _NSUB * _ZPT)])

    def flush(rng):
        obase = (c * (_NRANGE - 1) + rng - 1) * _RN
        pltpu.sync_copy(ab_sh.at[pl.ds(s * _ZPT, _ZPT)],
                        ab_out.at[pl.ds(obase + s * _ZPT, _ZPT)])

        @pl.when(s == _NSUB - 1)
        def _():
            pltpu.sync_copy(
                ab_sh.at[pl.ds(_NSUB * _ZPT, _RN - _NSUB * _ZPT)],
                ab_out.at[pl.ds(obase + _NSUB * _ZPT, _RN - _NSUB * _ZPT)])

    zero_acc()
    plsc.subcore_barrier()

    for rng in range(1, _NRANGE):
        ibase = (rng - 1) * _E

        def rin_issue(k, b):
            base = base0 + k * _C
            pltpu.async_copy(iic_hbm.at[pl.ds(ibase + base, _C)], IS[b],
                             ISEM[b])
            pltpu.async_copy(msig_hbm.at[pl.ds(c * _E + base, _C)], MF[b],
                             ISEM[b])

        def rin_wait(k, b):
            base = base0 + k * _C
            pltpu.make_async_copy(iic_hbm.at[pl.ds(ibase + base, _C)], IS[b],
                                  ISEM[b]).wait()
            pltpu.make_async_copy(msig_hbm.at[pl.ds(c * _E + base, _C)], MF[b],
                                  ISEM[b]).wait()

        rin_issue(0, 0)
        rin_issue(1, 1)

        def body2(k2, carry):
            k0 = 2 * k2
            k1 = k0 + 1
            rin_wait(k0, 0)
            d0 = pltpu.async_copy(MF[0], ab_sh.at[IS[0]], OSEM[0], add=True)
            rin_wait(k1, 1)
            d1 = pltpu.async_copy(MF[1], ab_sh.at[IS[1]], OSEM[1], add=True)
            d0.wait()

            @pl.when(k0 + 2 < _NCH)
            def _():
                rin_issue(k0 + 2, 0)

            d1.wait()

            @pl.when(k1 + 2 < _NCH)
            def _():
                rin_issue(k1 + 2, 1)
            return carry

        lax.fori_loop(0, _NCH // 2, body2, 0)
        plsc.subcore_barrier()
        flush(rng)
        if rng < _NRANGE - 1:
            plsc.subcore_barrier()
            zero_acc()
            plsc.subcore_barrier()


def _node_fin1_body(a_ref, b_ref, xu_ref, xpre_ref, nstats_ref, acc):
    i = pl.program_id(0)

    @pl.when(i == 0)
    def _():
        acc[...] = jnp.zeros((2, _D), jnp.float32)

    xp = xu_ref[...] + a_ref[...] / (b_ref[...] + 1e-6)
    xpre_ref[...] = xp
    acc[0:1, :] = acc[0:1, :] + jnp.sum(xp, axis=0, keepdims=True)
    acc[1:2, :] = acc[1:2, :] + jnp.sum(xp * xp, axis=0, keepdims=True)
    nstats_ref[...] = acc[...]


def _node_fin2_body(xpre_ref, nf_ref, nstats_ref, gamma_ref, beta_ref, out_ref):
    mean = nstats_ref[0:1, :] / _N
    var = nstats_ref[1:2, :] / _N - mean * mean
    xp = xpre_ref[...]
    xn = (xp - mean) * lax.rsqrt(var + 1e-5) * gamma_ref[...] + beta_ref[...]
    out_ref[...] = nf_ref[...] + xn * jax.nn.sigmoid(xn)


def _edge_fin_body(y_ref, ef_ref, stats_ref, gamma_ref, beta_ref, out_ref):
    st = stats_ref[...].reshape(2, _NSUB, 8, _H)
    sums = jnp.sum(st, axis=1)
    halves = []
    for cc in range(2):
        mean = (sums[cc, 0, :] / _E).reshape(1, _H)
        var = (sums[cc, 1, :] / _E).reshape(1, _H) - mean * mean
        yv = y_ref[cc]
        g = gamma_ref[:, cc * _H:(cc + 1) * _H]
        bt = beta_ref[:, cc * _H:(cc + 1) * _H]
        yn = (yv - mean) * lax.rsqrt(var + 1e-5) * g + bt
        halves.append(yn * jax.nn.sigmoid(yn))
    out_ref[...] = ef_ref[...] + jnp.concatenate(halves, axis=1)


@jax.jit
def kernel(node_feats, edge_feats, edge_index, W_sg, b_sg, W_dg, b_dg,
           W_eg, b_eg, W_su, b_su, W_du, b_du, gamma_n, beta_n,
           gamma_e, beta_e):
    f32 = jnp.float32
    b_sg2 = b_sg.reshape(1, _D)
    b_dg2 = b_dg.reshape(1, _D)
    b_du2 = b_du.reshape(1, _D)
    b_su2 = b_su.reshape(1, _D)
    b_eg2 = b_eg.reshape(1, _D)

    n_grid = _N // _NBLK
    s_tab, du_tab, xu_tab = pl.pallas_call(
        _node_prep_body,
        grid=(n_grid,),
        in_specs=[
            pl.BlockSpec((_NBLK, _D), lambda i: (i, 0)),
            pl.BlockSpec((_D, _D), lambda i: (0, 0)),
            pl.BlockSpec((_D, _D), lambda i: (0, 0)),
            pl.BlockSpec((_D, _D), lambda i: (0, 0)),
            pl.BlockSpec((_D, _D), lambda i: (0, 0)),
            pl.BlockSpec((1, _D), lambda i: (0, 0)),
            pl.BlockSpec((1, _D), lambda i: (0, 0)),
            pl.BlockSpec((1, _D), lambda i: (0, 0)),
            pl.BlockSpec((1, _D), lambda i: (0, 0)),
        ],
        out_specs=[
            pl.BlockSpec((_NBLK, _D), lambda i: (i, 0)),
            pl.BlockSpec((2, _NBLK, _D), lambda i: (0, i, 0)),
            pl.BlockSpec((_NBLK, _D), lambda i: (i, 0)),
        ],
        out_shape=[
            jax.ShapeDtypeStruct((_N, _D), f32),
            jax.ShapeDtypeStruct((2, _N, _D), f32),
            jax.ShapeDtypeStruct((_N, _D), f32),
        ],
    )(node_feats, W_sg, W_dg, W_du, W_su, b_sg2, b_dg2, b_du2, b_su2)

    e_grid = _E // _EBLK
    eg_tab = pl.pallas_call(
        _edge_prep_body,
        grid=(e_grid,),
        in_specs=[
            pl.BlockSpec((_EBLK, _D), lambda i: (i, 0)),
            pl.BlockSpec((_D, _D), lambda i: (0, 0)),
            pl.BlockSpec((1, _D), lambda i: (0, 0)),
        ],
        out_specs=[pl.BlockSpec((2, _EBLK, _H), lambda i: (0, i, 0))],
        out_shape=[jax.ShapeDtypeStruct((2, _E, _H), f32)],
    )(edge_feats, W_eg, b_eg2)[0]

    scat = s_tab
    ducat = du_tab.reshape(2 * _N, _D)
    eeg = eg_tab.reshape(2 * _E, _H)
    idx_i = edge_index[0]
    idx_j = edge_index[1]
    jjcat = jnp.concatenate([idx_j, idx_j + _N])
    iic = []
    for rng in range(1, _NRANGE):
        lo = rng * _RN
        inr = jnp.logical_and(idx_i >= lo, idx_i < lo + _RN)
        iic.append(jnp.where(inr, idx_i - lo, _JUNK))
    iicat = jnp.concatenate(iic)

    mesh = plsc.VectorSubcoreMesh(core_axis_name="c", subcore_axis_name="s")
    sc_edge = functools.partial(
        pl.kernel,
        mesh=mesh,
        compiler_params=pltpu.CompilerParams(needs_layout_passes=False),
        out_type=[
            jax.ShapeDtypeStruct((2 * _E, _H), f32),
            jax.ShapeDtypeStruct((2 * _E, _D), f32),
            jax.ShapeDtypeStruct((2 * _RN, _D), f32),
            jax.ShapeDtypeStruct((2 * _NSUB * 8, _H), f32),
        ],
        scratch_types=(
            [pltpu.VMEM((_C,), jnp.int32)] * 6
            + [pltpu.VMEM((_C, _D), f32)] * 4
            + [pltpu.VMEM((_C, _H), f32)] * 4
            + [pltpu.VMEM((_C, _D), f32)] * 2
            + [pltpu.VMEM((8, _H), f32)]
            + [pltpu.VMEM_SHARED((_AROWS, _D), f32)]
            + [pltpu.SemaphoreType.DMA] * 6
        ),
    )(_sc_edge_body)
    y_flat, msig, ab0_flat, sc_stats = sc_edge(
        scat, ducat, eeg, idx_i, jjcat, iicat)

    sc_relay = functools.partial(
        pl.kernel,
        mesh=mesh,
        compiler_params=pltpu.CompilerParams(needs_layout_passes=False),
        out_type=[
            jax.ShapeDtypeStruct((2 * (_NRANGE - 1) * _RN, _D), f32),
        ],
        scratch_types=(
            [pltpu.VMEM((_C,), jnp.int32)] * 2
            + [pltpu.VMEM((_C, _D), f32)] * 3
            + [pltpu.VMEM_SHARED((_AROWS, _D), f32)]
            + [pltpu.SemaphoreType.DMA] * 4
        ),
    )(_sc_relay_body)
    ab12_flat = sc_relay(msig, iicat)[0]

    ab0 = ab0_flat.reshape(2, _RN, _D)
    ab12 = ab12_flat.reshape(2, (_NRANGE - 1) * _RN, _D)
    ab = jnp.concatenate([ab0, ab12], axis=1)[:, :_N, :]
    a_tab = jnp.concatenate([ab[0, :, :_H], ab[1, :, :_H]], axis=1)
    b_tab = jnp.concatenate([ab[0, :, _H:], ab[1, :, _H:]], axis=1)
    y_tab = y_flat.reshape(2, _E, _H)

    xpre, nstats = pl.pallas_call(
        _node_fin1_body,
        grid=(n_grid,),
        in_specs=[
            pl.BlockSpec((_NBLK, _D), lambda i: (i, 0)),
            pl.BlockSpec((_NBLK, _D), lambda i: (i, 0)),
            pl.BlockSpec((_NBLK, _D), lambda i: (i, 0)),
        ],
        out_specs=[
            pl.BlockSpec((_NBLK, _D), lambda i: (i, 0)),
            pl.BlockSpec((2, _D), lambda i: (0, 0)),
        ],
        out_shape=[
            jax.ShapeDtypeStruct((_N, _D), f32),
            jax.ShapeDtypeStruct((2, _D), f32),
        ],
        scratch_shapes=[pltpu.VMEM((2, _D), f32)],
    )(a_tab, b_tab, xu_tab)

    x_out = pl.pallas_call(
        _node_fin2_body,
        grid=(n_grid,),
        in_specs=[
            pl.BlockSpec((_NBLK, _D), lambda i: (i, 0)),
            pl.BlockSpec((_NBLK, _D), lambda i: (i, 0)),
            pl.BlockSpec((2, _D), lambda i: (0, 0)),
            pl.BlockSpec((1, _D), lambda i: (0, 0)),
            pl.BlockSpec((1, _D), lambda i: (0, 0)),
        ],
        out_specs=pl.BlockSpec((_NBLK, _D), lambda i: (i, 0)),
        out_shape=jax.ShapeDtypeStruct((_N, _D), f32),
    )(xpre, node_feats, nstats, gamma_n.reshape(1, _D), beta_n.reshape(1, _D))

    y_out = pl.pallas_call(
        _edge_fin_body,
        grid=(e_grid,),
        in_specs=[
            pl.BlockSpec((2, _EBLK, _H), lambda i: (0, i, 0)),
            pl.BlockSpec((_EBLK, _D), lambda i: (i, 0)),
            pl.BlockSpec((2 * _NSUB * 8, _H), lambda i: (0, 0)),
            pl.BlockSpec((1, _D), lambda i: (0, 0)),
            pl.BlockSpec((1, _D), lambda i: (0, 0)),
        ],
        out_specs=pl.BlockSpec((_EBLK, _D), lambda i: (i, 0)),
        out_shape=jax.ShapeDtypeStruct((_E, _D), f32),
    )(y_tab, edge_feats, sc_stats, gamma_e.reshape(1, _D), beta_e.reshape(1, _D))

    return (x_out, y_out)
